# baseline (speedup 1.0000x reference)
import functools

import jax
import jax.numpy as jnp
from jax import lax
from jax.experimental import pallas as pl
from jax.experimental.pallas import tpu as pltpu
from jax.experimental.pallas import tpu_sc as plsc

D = 64
DP = 128
L = 16
NC, NS = 2, 16
NW = NC * NS
G = 128
GPC = 2
K = G * GPC


@functools.lru_cache(maxsize=None)
def _build(ntok):
    n_per_w = ntok // NW
    nchunks = n_per_w // K
    assert ntok % NW == 0 and n_per_w % K == 0
    mesh = plsc.VectorSubcoreMesh(
        core_axis_name="c", subcore_axis_name="s", num_cores=NC, num_subcores=NS
    )

    @functools.partial(
        pl.kernel,
        out_type=jax.ShapeDtypeStruct((ntok, D), jnp.float32),
        mesh=mesh,
        scratch_types=[
            pltpu.VMEM((GPC, G), jnp.int32),
            pltpu.VMEM((GPC, G), jnp.int32),
            pltpu.VMEM((K, DP), jnp.float32),
            pltpu.VMEM((K, DP), jnp.float32),
            pltpu.VMEM((K, D), jnp.float32),
            pltpu.SemaphoreType.DMA,
        ],
        compiler_params=pltpu.CompilerParams(use_tc_tiling_on_sc=True),
    )
    def sc_kernel(ids_hbm, ca_hbm, cent_hbm, off_hbm, out_hbm,
                  ids_v, cids_v, big_v, cen_v, acc_v, sem):
        wid = lax.axis_index("s") * NC + lax.axis_index("c")
        wbase = wid * n_per_w

        @pl.loop(0, nchunks)
        def _chunk(c):
            base = wbase + c * K
            for j in range(GPC):
                pltpu.sync_copy(ids_hbm.at[pl.ds(base + j * G, G)], ids_v.at[j])
            cid_descs = [
                pltpu.async_copy(ca_hbm.at[ids_v.at[j]], cids_v.at[j], sem)
                for j in range(GPC)
            ]
            big_descs = [
                pltpu.async_copy(
                    off_hbm.at[ids_v.at[j]], big_v.at[pl.ds(j * G, G)], sem
                )
                for j in range(GPC)
            ]
            for dsc in cid_descs:
                dsc.wait()
            cen_descs = [
                pltpu.async_copy(
                    cent_hbm.at[cids_v.at[j]], cen_v.at[pl.ds(j * G, G)], sem
                )
                for j in range(GPC)
            ]
            for dsc in big_descs + cen_descs:
                dsc.wait()

            @pl.loop(0, K)
            def _add(t):
                for d in range(D // L):
                    sl = pl.ds(d * L, L)
                    acc_v[t, sl] = big_v[t, sl] + cen_v[t, sl]

            pltpu.sync_copy(acc_v, out_hbm.at[pl.ds(base, K)])

    return sc_kernel


def kernel(input_ids, cluster_assign, centroids, offsets):
    b, t = input_ids.shape
    ids = input_ids.reshape(-1)
    off_p = jnp.pad(offsets, ((0, 0), (0, DP - D)))
    cen_p = jnp.pad(centroids, ((0, 0), (0, DP - D)))
    out = _build(ids.shape[0])(ids, cluster_assign, cen_p, off_p)
    return out.reshape(b, t, D)
